# baseline (speedup 1.0000x reference)
import functools

import jax
import jax.numpy as jnp
from jax import lax
from jax.experimental import pallas as pl
from jax.experimental.pallas import tpu as pltpu
from jax.experimental.pallas import tpu_sc as plsc

N_NODES = 10000
N_EDGES = 320000
D_IN = 128
D_HID = 128
D_OUT = 64

NC = 2
NS = 16
NW = NC * NS
E_PER_TILE = N_EDGES // NW
N_PAD = 10240
ROWS_PER_TILE = N_PAD // NS

D1P = D_HID + 16
D2P = D_OUT + 8

NBUF = 5

_sc_mesh = plsc.VectorSubcoreMesh(core_axis_name="c", subcore_axis_name="s")
_sc_params = pltpu.CompilerParams(use_tc_tiling_on_sc=False)


def _make_sc_agg(d, chunk):
    n_chunks = E_PER_TILE // chunk
    n_groups = n_chunks // NBUF

    @functools.partial(
        pl.kernel,
        mesh=_sc_mesh,
        compiler_params=_sc_params,
        out_type=jax.ShapeDtypeStruct((NC, N_PAD, d), jnp.bfloat16),
        scratch_types=[
            pltpu.VMEM((n_chunks, chunk), jnp.int32),
            pltpu.VMEM((n_chunks, chunk), jnp.int32),
            pltpu.VMEM((NBUF, chunk, d), jnp.bfloat16),
            pltpu.VMEM_SHARED((N_PAD, d), jnp.bfloat16),
            pltpu.SemaphoreType.DMA((NBUF,)),
            pltpu.SemaphoreType.DMA((NBUF,)),
        ],
    )
    def sc_agg(z_hbm, ei_hbm, zinit_hbm, out_hbm,
               src_v, dst_v, rows_v, acc_sh, gsem, ssem):
        cid = lax.axis_index("c")
        sid = lax.axis_index("s")
        wid = cid * NS + sid

        pltpu.sync_copy(ei_hbm.at[0, wid], src_v)
        pltpu.sync_copy(ei_hbm.at[1, wid], dst_v)
        pltpu.sync_copy(zinit_hbm,
                        acc_sh.at[pl.ds(sid * ROWS_PER_TILE, ROWS_PER_TILE), :])
        plsc.subcore_barrier()

        for t in range(NBUF):
            pltpu.async_copy(z_hbm.at[src_v.at[t]], rows_v.at[t], gsem.at[t])

        def body(g, carry):
            j0 = g * NBUF
            for t in range(NBUF):
                j = j0 + t
                pltpu.make_async_copy(z_hbm.at[src_v.at[j]], rows_v.at[t],
                                      gsem.at[t]).wait()
                pltpu.async_copy(rows_v.at[t], acc_sh.at[dst_v.at[j]],
                                 ssem.at[t], add=True)
            for t in range(NBUF):
                j = j0 + t
                jn = j + NBUF
                pltpu.make_async_copy(rows_v.at[t], acc_sh.at[dst_v.at[j]],
                                      ssem.at[t]).wait()

                @pl.when(jn < n_chunks)
                def _():
                    pltpu.async_copy(z_hbm.at[src_v.at[jn]], rows_v.at[t],
                                     gsem.at[t])
            return carry

        lax.fori_loop(0, n_groups, body, 0)
        plsc.subcore_barrier()

        pltpu.sync_copy(acc_sh.at[pl.ds(sid * ROWS_PER_TILE, ROWS_PER_TILE), :],
                        out_hbm.at[cid, pl.ds(sid * ROWS_PER_TILE, ROWS_PER_TILE), :])

    return sc_agg


_CHUNK = 80
_N_CHUNKS = E_PER_TILE // _CHUNK
_sc_agg1 = _make_sc_agg(D1P, _CHUNK)
_sc_agg2 = _make_sc_agg(D_OUT, _CHUNK)


_RB = 1000
_NRB = N_NODES // _RB


def _tc1a_body(x_ref, w1l_ref, z1_ref):
    z1 = jnp.dot(x_ref[...], w1l_ref[...], preferred_element_type=jnp.float32)
    ones = jnp.ones((_RB, 1), jnp.float32)
    pad = jnp.zeros((_RB, D1P - D_HID - 1), jnp.float32)
    z1_ref[...] = jnp.concatenate([z1, ones, pad], axis=1).astype(jnp.bfloat16)


def _tc1a(x, w1l):
    return pl.pallas_call(
        _tc1a_body,
        grid=(_NRB,),
        in_specs=[
            pl.BlockSpec((_RB, D_IN), lambda i: (i, 0)),
            pl.BlockSpec((D_IN, D_HID), lambda i: (0, 0)),
        ],
        out_specs=pl.BlockSpec((_RB, D1P), lambda i: (i, 0)),
        out_shape=jax.ShapeDtypeStruct((N_NODES, D1P), jnp.bfloat16),
    )(x, w1l)


def _tc1b_body(x_ref, w1r_ref, b1r_ref, r1_ref):
    r1_ref[...] = (jnp.dot(x_ref[...], w1r_ref[...],
                           preferred_element_type=jnp.float32) + b1r_ref[...])


def _tc1b(x, w1r, b1r):
    return pl.pallas_call(
        _tc1b_body,
        grid=(_NRB,),
        in_specs=[
            pl.BlockSpec((_RB, D_IN), lambda i: (i, 0)),
            pl.BlockSpec((D_IN, D_HID), lambda i: (0, 0)),
            pl.BlockSpec((1, D_HID), lambda i: (0, 0)),
        ],
        out_specs=pl.BlockSpec((_RB, D_HID), lambda i: (i, 0)),
        out_shape=jax.ShapeDtypeStruct((N_NODES, D_HID), jnp.float32),
    )(x, w1r, b1r)


def _tc2a_body(p1_ref, r1_ref, b1l_ref, w2l_ref, h_ref, z2_ref):
    s1 = (p1_ref[0].astype(jnp.float32)
          + p1_ref[1].astype(jnp.float32))
    agg = s1[:, :D_HID]
    deg = s1[:, D_HID:D_HID + 1]
    invd = 1.0 / jnp.maximum(deg, 1.0)
    h = jnp.maximum(agg * invd + b1l_ref[...] + r1_ref[...], 0.0)
    h_ref[...] = h
    z2 = jnp.dot(h, w2l_ref[...], preferred_element_type=jnp.float32)
    z2_ref[...] = z2.astype(jnp.bfloat16)


def _tc2a(p1, r1, b1l, w2l):
    return pl.pallas_call(
        _tc2a_body,
        grid=(_NRB,),
        in_specs=[
            pl.BlockSpec((NC, _RB, D1P), lambda i: (0, i, 0)),
            pl.BlockSpec((_RB, D_HID), lambda i: (i, 0)),
            pl.BlockSpec((1, D_HID), lambda i: (0, 0)),
            pl.BlockSpec((D_HID, D_OUT), lambda i: (0, 0)),
        ],
        out_specs=[
            pl.BlockSpec((_RB, D_HID), lambda i: (i, 0)),
            pl.BlockSpec((_RB, D_OUT), lambda i: (i, 0)),
        ],
        out_shape=[
            jax.ShapeDtypeStruct((N_NODES, D_HID), jnp.float32),
            jax.ShapeDtypeStruct((N_NODES, D_OUT), jnp.bfloat16),
        ],
    )(p1, r1, b1l, w2l)


def _tc2b_body(h_ref, p1_ref, w2r_ref, b2c_ref, r2p_ref):
    deg = (p1_ref[0, :, D_HID:D_HID + 1].astype(jnp.float32)
           + p1_ref[1, :, D_HID:D_HID + 1].astype(jnp.float32))
    invd = 1.0 / jnp.maximum(deg, 1.0)
    r2c = (jnp.dot(h_ref[...], w2r_ref[...], preferred_element_type=jnp.float32)
           + b2c_ref[...])
    pad = jnp.zeros((_RB, D2P - D_OUT - 1), jnp.float32)
    r2p_ref[...] = jnp.concatenate([r2c, invd, pad], axis=1)


def _tc2b(h, p1, w2r, b2c):
    return pl.pallas_call(
        _tc2b_body,
        grid=(_NRB,),
        in_specs=[
            pl.BlockSpec((_RB, D_HID), lambda i: (i, 0)),
            pl.BlockSpec((NC, _RB, D1P), lambda i: (0, i, 0)),
            pl.BlockSpec((D_HID, D_OUT), lambda i: (0, 0)),
            pl.BlockSpec((1, D_OUT), lambda i: (0, 0)),
        ],
        out_specs=pl.BlockSpec((_RB, D2P), lambda i: (i, 0)),
        out_shape=jax.ShapeDtypeStruct((N_NODES, D2P), jnp.float32),
    )(h, p1, w2r, b2c)


def _tc3_body(p2_ref, r2p_ref, y_ref, m_ref, out_ref, num_ref, den_ref):
    i = pl.program_id(0)

    agg2 = (p2_ref[0].astype(jnp.float32)
            + p2_ref[1].astype(jnp.float32))
    r2c = r2p_ref[:, :D_OUT]
    invd = r2p_ref[:, D_OUT:D_OUT + 1]
    logits = agg2 * invd + r2c
    mx = jnp.max(logits, axis=1, keepdims=True)
    lse = jnp.log(jnp.sum(jnp.exp(logits - mx), axis=1, keepdims=True))
    lsm = logits - mx - lse
    onehot = (lax.broadcasted_iota(jnp.int32, (_RB, D_OUT), 1)
              == y_ref[...]).astype(jnp.float32)
    picked = jnp.sum(lsm * onehot, axis=1, keepdims=True)
    m = m_ref[...]
    num_p = jnp.sum(picked * m)
    den_p = jnp.sum(m)

    @pl.when(i == 0)
    def _():
        num_ref[0] = num_p
        den_ref[0] = den_p

    @pl.when(i > 0)
    def _():
        num_ref[0] = num_ref[0] + num_p
        den_ref[0] = den_ref[0] + den_p

    @pl.when(i == _NRB - 1)
    def _():
        loss = -num_ref[0] / jnp.maximum(den_ref[0], 1.0)
        out_ref[...] = jnp.broadcast_to(loss, (1, 1))


def _tc3(p2, r2p, y2d, m2d):
    return pl.pallas_call(
        _tc3_body,
        grid=(_NRB,),
        in_specs=[
            pl.BlockSpec((NC, _RB, D_OUT), lambda i: (0, i, 0)),
            pl.BlockSpec((_RB, D2P), lambda i: (i, 0)),
            pl.BlockSpec((_RB, 1), lambda i: (i, 0)),
            pl.BlockSpec((_RB, 1), lambda i: (i, 0)),
        ],
        out_specs=pl.BlockSpec((1, 1), lambda i: (0, 0)),
        out_shape=jax.ShapeDtypeStruct((1, 1), jnp.float32),
        scratch_shapes=[
            pltpu.SMEM((1,), jnp.float32),
            pltpu.SMEM((1,), jnp.float32),
        ],
    )(p2, r2p, y2d, m2d)


def kernel(x, edge_index, y, train_mask,
           W1_l, b1_l, W1_r, b1_r, W2_l, b2_l, W2_r, b2_r):
    ei = edge_index.reshape(2, NW, _N_CHUNKS, _CHUNK)
    zinit1 = jnp.zeros((ROWS_PER_TILE, D1P), jnp.bfloat16)
    zinit2 = jnp.zeros((ROWS_PER_TILE, D_OUT), jnp.bfloat16)

    z1 = _tc1a(x, W1_l)
    p1 = _sc_agg1(z1, ei, zinit1)
    r1 = _tc1b(x, W1_r, b1_r.reshape(1, D_HID))
    h, z2 = _tc2a(p1, r1, b1_l.reshape(1, D_HID), W2_l)
    p2 = _sc_agg2(z2, ei, zinit2)
    b2c = (b2_l + b2_r).reshape(1, D_OUT)
    r2p = _tc2b(h, p1, W2_r, b2c)
    loss = _tc3(p2, r2p, y.reshape(N_NODES, 1).astype(jnp.int32),
                train_mask.reshape(N_NODES, 1).astype(jnp.float32))
    return loss.reshape(1)


# speedup vs baseline: 1.0922x; 1.0922x over previous
import functools

import jax
import jax.numpy as jnp
from jax import lax
from jax.experimental import pallas as pl
from jax.experimental.pallas import tpu as pltpu
from jax.experimental.pallas import tpu_sc as plsc

N_NODES = 10000
N_EDGES = 320000
D_IN = 128
D_HID = 128
D_OUT = 64

NC = 2
NS = 16
NW = NC * NS
E_PER_TILE = N_EDGES // NW
N_PAD = 10240
ROWS_PER_TILE = N_PAD // NS

D2P = D_OUT + 8
DDEG = 8

NBUF = 5

_sc_mesh = plsc.VectorSubcoreMesh(core_axis_name="c", subcore_axis_name="s")
_sc_params = pltpu.CompilerParams(use_tc_tiling_on_sc=False)


def _make_sc_agg(d, chunk):
    n_chunks = E_PER_TILE // chunk
    n_groups = n_chunks // NBUF

    @functools.partial(
        pl.kernel,
        mesh=_sc_mesh,
        compiler_params=_sc_params,
        out_type=jax.ShapeDtypeStruct((NC, N_PAD, d), jnp.bfloat16),
        scratch_types=[
            pltpu.VMEM((n_chunks, chunk), jnp.int32),
            pltpu.VMEM((n_chunks, chunk), jnp.int32),
            pltpu.VMEM((NBUF, chunk, d), jnp.bfloat16),
            pltpu.VMEM_SHARED((N_PAD, d), jnp.bfloat16),
            pltpu.SemaphoreType.DMA((NBUF,)),
            pltpu.SemaphoreType.DMA((NBUF,)),
        ],
    )
    def sc_agg(z_hbm, ei_hbm, zinit_hbm, out_hbm,
               src_v, dst_v, rows_v, acc_sh, gsem, ssem):
        cid = lax.axis_index("c")
        sid = lax.axis_index("s")
        wid = cid * NS + sid

        pltpu.sync_copy(ei_hbm.at[0, wid], src_v)
        pltpu.sync_copy(ei_hbm.at[1, wid], dst_v)
        pltpu.sync_copy(zinit_hbm,
                        acc_sh.at[pl.ds(sid * ROWS_PER_TILE, ROWS_PER_TILE), :])
        plsc.subcore_barrier()

        for t in range(NBUF):
            pltpu.async_copy(z_hbm.at[src_v.at[t]], rows_v.at[t], gsem.at[t])

        def body(g, carry):
            j0 = g * NBUF
            for t in range(NBUF):
                j = j0 + t
                pltpu.make_async_copy(z_hbm.at[src_v.at[j]], rows_v.at[t],
                                      gsem.at[t]).wait()
                pltpu.async_copy(rows_v.at[t], acc_sh.at[dst_v.at[j]],
                                 ssem.at[t], add=True)
            for t in range(NBUF):
                j = j0 + t
                jn = j + NBUF
                pltpu.make_async_copy(rows_v.at[t], acc_sh.at[dst_v.at[j]],
                                      ssem.at[t]).wait()

                @pl.when(jn < n_chunks)
                def _():
                    pltpu.async_copy(z_hbm.at[src_v.at[jn]], rows_v.at[t],
                                     gsem.at[t])
            return carry

        lax.fori_loop(0, n_groups, body, 0)
        plsc.subcore_barrier()

        pltpu.sync_copy(acc_sh.at[pl.ds(sid * ROWS_PER_TILE, ROWS_PER_TILE), :],
                        out_hbm.at[cid, pl.ds(sid * ROWS_PER_TILE, ROWS_PER_TILE), :])

    return sc_agg


_CHUNK = 80
_N_CHUNKS = E_PER_TILE // _CHUNK
_N_GROUPS = _N_CHUNKS // NBUF
_sc_agg1 = _make_sc_agg(D_HID, _CHUNK)
_sc_agg2 = _make_sc_agg(D_OUT, _CHUNK)


@functools.partial(
    pl.kernel,
    mesh=_sc_mesh,
    compiler_params=_sc_params,
    out_type=jax.ShapeDtypeStruct((NC, N_PAD, DDEG), jnp.float32),
    scratch_types=[
        pltpu.VMEM((_N_CHUNKS, _CHUNK), jnp.int32),
        pltpu.VMEM((_CHUNK, DDEG), jnp.float32),
        pltpu.VMEM_SHARED((N_PAD, DDEG), jnp.float32),
        pltpu.SemaphoreType.DMA((NBUF,)),
    ],
)
def _sc_deg(ei_hbm, ones_hbm, zinit_hbm, out_hbm, dst_v, ones_v, acc_sh, ssem):
    cid = lax.axis_index("c")
    sid = lax.axis_index("s")
    wid = cid * NS + sid

    pltpu.sync_copy(ei_hbm.at[1, wid], dst_v)
    pltpu.sync_copy(ones_hbm, ones_v)
    pltpu.sync_copy(zinit_hbm,
                    acc_sh.at[pl.ds(sid * ROWS_PER_TILE, ROWS_PER_TILE), :])
    plsc.subcore_barrier()

    def body(g, carry):
        j0 = g * NBUF
        for t in range(NBUF):
            pltpu.async_copy(ones_v, acc_sh.at[dst_v.at[j0 + t]],
                             ssem.at[t], add=True)
        for t in range(NBUF):
            pltpu.make_async_copy(ones_v, acc_sh.at[dst_v.at[j0 + t]],
                                  ssem.at[t]).wait()
        return carry

    lax.fori_loop(0, _N_GROUPS, body, 0)
    plsc.subcore_barrier()

    pltpu.sync_copy(acc_sh.at[pl.ds(sid * ROWS_PER_TILE, ROWS_PER_TILE), :],
                    out_hbm.at[cid, pl.ds(sid * ROWS_PER_TILE, ROWS_PER_TILE), :])


_RB = 1000
_NRB = N_NODES // _RB


def _tc1a_body(x_ref, w1l_ref, z1_ref):
    z1 = jnp.dot(x_ref[...].astype(jnp.bfloat16),
                 w1l_ref[...].astype(jnp.bfloat16),
                 preferred_element_type=jnp.float32)
    z1_ref[...] = z1.astype(jnp.bfloat16)


def _tc1a(x, w1l):
    return pl.pallas_call(
        _tc1a_body,
        grid=(_NRB,),
        in_specs=[
            pl.BlockSpec((_RB, D_IN), lambda i: (i, 0)),
            pl.BlockSpec((D_IN, D_HID), lambda i: (0, 0)),
        ],
        out_specs=pl.BlockSpec((_RB, D_HID), lambda i: (i, 0)),
        out_shape=jax.ShapeDtypeStruct((N_NODES, D_HID), jnp.bfloat16),
    )(x, w1l)


def _tc1b_body(x_ref, w1r_ref, b1r_ref, ds_ref, r1_ref):
    r1 = (jnp.dot(x_ref[...].astype(jnp.bfloat16),
                  w1r_ref[...].astype(jnp.bfloat16),
                  preferred_element_type=jnp.float32) + b1r_ref[...])
    r1_ref[...] = r1 + ds_ref[0, 0, 0] * 0.0


def _tc1b(x, w1r, b1r, dsum):
    return pl.pallas_call(
        _tc1b_body,
        grid=(_NRB,),
        in_specs=[
            pl.BlockSpec((_RB, D_IN), lambda i: (i, 0)),
            pl.BlockSpec((D_IN, D_HID), lambda i: (0, 0)),
            pl.BlockSpec((1, D_HID), lambda i: (0, 0)),
            pl.BlockSpec((NC, 8, DDEG), lambda i: (0, 0, 0)),
        ],
        out_specs=pl.BlockSpec((_RB, D_HID), lambda i: (i, 0)),
        out_shape=jax.ShapeDtypeStruct((N_NODES, D_HID), jnp.float32),
    )(x, w1r, b1r, dsum)


def _tc2a_body(p1_ref, ds_ref, r1_ref, b1l_ref, w2l_ref, h_ref, z2_ref):
    agg = (p1_ref[0].astype(jnp.float32)
           + p1_ref[1].astype(jnp.float32))
    ds = ds_ref[0] + ds_ref[1]
    deg = ds[:, 0:1]
    invd = 1.0 / jnp.maximum(deg, 1.0)
    h = jnp.maximum(agg * invd + b1l_ref[...] + r1_ref[...], 0.0)
    h_ref[...] = h
    z2 = jnp.dot(h.astype(jnp.bfloat16), w2l_ref[...].astype(jnp.bfloat16),
                 preferred_element_type=jnp.float32)
    z2_ref[...] = z2.astype(jnp.bfloat16)


def _tc2a(p1, dsum, r1, b1l, w2l):
    return pl.pallas_call(
        _tc2a_body,
        grid=(_NRB,),
        in_specs=[
            pl.BlockSpec((NC, _RB, D_HID), lambda i: (0, i, 0)),
            pl.BlockSpec((NC, _RB, DDEG), lambda i: (0, i, 0)),
            pl.BlockSpec((_RB, D_HID), lambda i: (i, 0)),
            pl.BlockSpec((1, D_HID), lambda i: (0, 0)),
            pl.BlockSpec((D_HID, D_OUT), lambda i: (0, 0)),
        ],
        out_specs=[
            pl.BlockSpec((_RB, D_HID), lambda i: (i, 0)),
            pl.BlockSpec((_RB, D_OUT), lambda i: (i, 0)),
        ],
        out_shape=[
            jax.ShapeDtypeStruct((N_NODES, D_HID), jnp.float32),
            jax.ShapeDtypeStruct((N_NODES, D_OUT), jnp.bfloat16),
        ],
    )(p1, dsum, r1, b1l, w2l)


def _tc2b_body(h_ref, ds_ref, w2r_ref, b2c_ref, r2p_ref):
    ds = ds_ref[0] + ds_ref[1]
    deg = ds[:, 0:1]
    invd = 1.0 / jnp.maximum(deg, 1.0)
    r2c = (jnp.dot(h_ref[...].astype(jnp.bfloat16),
                   w2r_ref[...].astype(jnp.bfloat16),
                   preferred_element_type=jnp.float32) + b2c_ref[...])
    pad = jnp.zeros((_RB, D2P - D_OUT - 1), jnp.float32)
    r2p_ref[...] = jnp.concatenate([r2c, invd, pad], axis=1)


def _tc2b(h, dsum, w2r, b2c):
    return pl.pallas_call(
        _tc2b_body,
        grid=(_NRB,),
        in_specs=[
            pl.BlockSpec((_RB, D_HID), lambda i: (i, 0)),
            pl.BlockSpec((NC, _RB, DDEG), lambda i: (0, i, 0)),
            pl.BlockSpec((D_HID, D_OUT), lambda i: (0, 0)),
            pl.BlockSpec((1, D_OUT), lambda i: (0, 0)),
        ],
        out_specs=pl.BlockSpec((_RB, D2P), lambda i: (i, 0)),
        out_shape=jax.ShapeDtypeStruct((N_NODES, D2P), jnp.float32),
    )(h, dsum, w2r, b2c)


def _tc3_body(p2_ref, r2p_ref, y_ref, m_ref, out_ref, num_ref, den_ref):
    i = pl.program_id(0)

    agg2 = (p2_ref[0].astype(jnp.float32)
            + p2_ref[1].astype(jnp.float32))
    r2c = r2p_ref[:, :D_OUT]
    invd = r2p_ref[:, D_OUT:D_OUT + 1]
    logits = agg2 * invd + r2c
    mx = jnp.max(logits, axis=1, keepdims=True)
    lse = jnp.log(jnp.sum(jnp.exp(logits - mx), axis=1, keepdims=True))
    lsm = logits - mx - lse
    onehot = (lax.broadcasted_iota(jnp.int32, (_RB, D_OUT), 1)
              == y_ref[...]).astype(jnp.float32)
    picked = jnp.sum(lsm * onehot, axis=1, keepdims=True)
    m = m_ref[...]
    num_p = jnp.sum(picked * m)
    den_p = jnp.sum(m)

    @pl.when(i == 0)
    def _():
        num_ref[0] = num_p
        den_ref[0] = den_p

    @pl.when(i > 0)
    def _():
        num_ref[0] = num_ref[0] + num_p
        den_ref[0] = den_ref[0] + den_p

    @pl.when(i == _NRB - 1)
    def _():
        loss = -num_ref[0] / jnp.maximum(den_ref[0], 1.0)
        out_ref[...] = jnp.broadcast_to(loss, (1, 1))


def _tc3(p2, r2p, y2d, m2d):
    return pl.pallas_call(
        _tc3_body,
        grid=(_NRB,),
        in_specs=[
            pl.BlockSpec((NC, _RB, D_OUT), lambda i: (0, i, 0)),
            pl.BlockSpec((_RB, D2P), lambda i: (i, 0)),
            pl.BlockSpec((_RB, 1), lambda i: (i, 0)),
            pl.BlockSpec((_RB, 1), lambda i: (i, 0)),
        ],
        out_specs=pl.BlockSpec((1, 1), lambda i: (0, 0)),
        out_shape=jax.ShapeDtypeStruct((1, 1), jnp.float32),
        scratch_shapes=[
            pltpu.SMEM((1,), jnp.float32),
            pltpu.SMEM((1,), jnp.float32),
        ],
    )(p2, r2p, y2d, m2d)


def kernel(x, edge_index, y, train_mask,
           W1_l, b1_l, W1_r, b1_r, W2_l, b2_l, W2_r, b2_r):
    ei = edge_index.reshape(2, NW, _N_CHUNKS, _CHUNK)
    zinit1 = jnp.zeros((ROWS_PER_TILE, D_HID), jnp.bfloat16)
    zinit2 = jnp.zeros((ROWS_PER_TILE, D_OUT), jnp.bfloat16)
    zinitd = jnp.zeros((ROWS_PER_TILE, DDEG), jnp.float32)
    onesd = jnp.ones((_CHUNK, DDEG), jnp.float32)

    dsum = _sc_deg(ei, onesd, zinitd)
    z1 = _tc1a(x, W1_l)
    p1 = _sc_agg1(z1, ei, zinit1)
    r1 = _tc1b(x, W1_r, b1_r.reshape(1, D_HID), dsum)
    h, z2 = _tc2a(p1, dsum, r1, b1_l.reshape(1, D_HID), W2_l)
    p2 = _sc_agg2(z2, ei, zinit2)
    b2c = (b2_l + b2_r).reshape(1, D_OUT)
    r2p = _tc2b(h, dsum, W2_r, b2c)
    loss = _tc3(p2, r2p, y.reshape(N_NODES, 1).astype(jnp.int32),
                train_mask.reshape(N_NODES, 1).astype(jnp.float32))
    return loss.reshape(1)


# speedup vs baseline: 1.1359x; 1.0400x over previous
import functools

import jax
import jax.numpy as jnp
from jax import lax
from jax.experimental import pallas as pl
from jax.experimental.pallas import tpu as pltpu
from jax.experimental.pallas import tpu_sc as plsc

N_NODES = 10000
N_EDGES = 320000
D_IN = 128
D_HID = 128
D_OUT = 64

NC = 2
NS = 16
NW = NC * NS
E_PER_TILE = N_EDGES // NW
N_PAD = 10240
ROWS_PER_TILE = N_PAD // NS

D2P = D_OUT + 8
DDEG = 8

NBUF = 5

_sc_mesh = plsc.VectorSubcoreMesh(core_axis_name="c", subcore_axis_name="s")
_sc_params = pltpu.CompilerParams(use_tc_tiling_on_sc=False)


def _make_sc_agg(d, chunk):
    n_chunks = E_PER_TILE // chunk
    n_groups = n_chunks // NBUF

    @functools.partial(
        pl.kernel,
        mesh=_sc_mesh,
        compiler_params=_sc_params,
        out_type=jax.ShapeDtypeStruct((NC, N_PAD, d), jnp.bfloat16),
        scratch_types=[
            pltpu.VMEM((n_chunks, chunk), jnp.int32),
            pltpu.VMEM((n_chunks, chunk), jnp.int32),
            pltpu.VMEM((NBUF, chunk, d), jnp.bfloat16),
            pltpu.VMEM_SHARED((N_PAD, d), jnp.bfloat16),
            pltpu.SemaphoreType.DMA((NBUF,)),
            pltpu.SemaphoreType.DMA((NBUF,)),
        ],
    )
    def sc_agg(z_hbm, ei_hbm, zinit_hbm, out_hbm,
               src_v, dst_v, rows_v, acc_sh, gsem, ssem):
        cid = lax.axis_index("c")
        sid = lax.axis_index("s")
        wid = cid * NS + sid

        pltpu.sync_copy(ei_hbm.at[0, wid], src_v)
        pltpu.sync_copy(ei_hbm.at[1, wid], dst_v)
        pltpu.sync_copy(zinit_hbm,
                        acc_sh.at[pl.ds(sid * ROWS_PER_TILE, ROWS_PER_TILE), :])
        plsc.subcore_barrier()

        for t in range(NBUF):
            pltpu.async_copy(z_hbm.at[src_v.at[t]], rows_v.at[t], gsem.at[t])

        def body(g, carry):
            j0 = g * NBUF
            for t in range(NBUF):
                j = j0 + t
                pltpu.make_async_copy(z_hbm.at[src_v.at[j]], rows_v.at[t],
                                      gsem.at[t]).wait()
                pltpu.async_copy(rows_v.at[t], acc_sh.at[dst_v.at[j]],
                                 ssem.at[t], add=True)
            for t in range(NBUF):
                j = j0 + t
                jn = j + NBUF
                pltpu.make_async_copy(rows_v.at[t], acc_sh.at[dst_v.at[j]],
                                      ssem.at[t]).wait()

                @pl.when(jn < n_chunks)
                def _():
                    pltpu.async_copy(z_hbm.at[src_v.at[jn]], rows_v.at[t],
                                     gsem.at[t])
            return carry

        lax.fori_loop(0, n_groups, body, 0)
        plsc.subcore_barrier()

        pltpu.sync_copy(acc_sh.at[pl.ds(sid * ROWS_PER_TILE, ROWS_PER_TILE), :],
                        out_hbm.at[cid, pl.ds(sid * ROWS_PER_TILE, ROWS_PER_TILE), :])

    return sc_agg


_CHUNK = 80
_N_CHUNKS = E_PER_TILE // _CHUNK
_N_GROUPS = _N_CHUNKS // NBUF
_sc_agg1 = _make_sc_agg(D_HID, _CHUNK)
_sc_agg2 = _make_sc_agg(D_OUT, _CHUNK)


@functools.partial(
    pl.kernel,
    mesh=_sc_mesh,
    compiler_params=_sc_params,
    out_type=jax.ShapeDtypeStruct((NC, N_PAD, DDEG), jnp.float32),
    scratch_types=[
        pltpu.VMEM((_N_CHUNKS, _CHUNK), jnp.int32),
        pltpu.VMEM((_CHUNK, DDEG), jnp.float32),
        pltpu.VMEM_SHARED((N_PAD, DDEG), jnp.float32),
        pltpu.SemaphoreType.DMA((NBUF,)),
    ],
)
def _sc_deg(ei_hbm, ones_hbm, zinit_hbm, out_hbm, dst_v, ones_v, acc_sh, ssem):
    cid = lax.axis_index("c")
    sid = lax.axis_index("s")
    wid = cid * NS + sid

    pltpu.sync_copy(ei_hbm.at[1, wid], dst_v)
    pltpu.sync_copy(ones_hbm, ones_v)
    pltpu.sync_copy(zinit_hbm,
                    acc_sh.at[pl.ds(sid * ROWS_PER_TILE, ROWS_PER_TILE), :])
    plsc.subcore_barrier()

    def body(g, carry):
        j0 = g * NBUF
        for t in range(NBUF):
            pltpu.async_copy(ones_v, acc_sh.at[dst_v.at[j0 + t]],
                             ssem.at[t], add=True)
        for t in range(NBUF):
            pltpu.make_async_copy(ones_v, acc_sh.at[dst_v.at[j0 + t]],
                                  ssem.at[t]).wait()
        return carry

    lax.fori_loop(0, _N_GROUPS, body, 0)
    plsc.subcore_barrier()

    pltpu.sync_copy(acc_sh.at[pl.ds(sid * ROWS_PER_TILE, ROWS_PER_TILE), :],
                    out_hbm.at[cid, pl.ds(sid * ROWS_PER_TILE, ROWS_PER_TILE), :])


_RB = 2000
_NRB = N_NODES // _RB


def _tc1a_body(x_ref, w1l_ref, z1_ref):
    z1 = jnp.dot(x_ref[...].astype(jnp.bfloat16),
                 w1l_ref[...].astype(jnp.bfloat16),
                 preferred_element_type=jnp.float32)
    z1_ref[...] = z1.astype(jnp.bfloat16)


def _tc1a(x, w1l):
    return pl.pallas_call(
        _tc1a_body,
        grid=(_NRB,),
        in_specs=[
            pl.BlockSpec((_RB, D_IN), lambda i: (i, 0)),
            pl.BlockSpec((D_IN, D_HID), lambda i: (0, 0)),
        ],
        out_specs=pl.BlockSpec((_RB, D_HID), lambda i: (i, 0)),
        out_shape=jax.ShapeDtypeStruct((N_NODES, D_HID), jnp.bfloat16),
    )(x, w1l)


def _tc1b_body(x_ref, w1r_ref, b1r_ref, ds_ref, r1_ref):
    r1 = (jnp.dot(x_ref[...].astype(jnp.bfloat16),
                  w1r_ref[...].astype(jnp.bfloat16),
                  preferred_element_type=jnp.float32) + b1r_ref[...])
    r1_ref[...] = r1 + ds_ref[0, 0, 0] * 0.0


def _tc1b(x, w1r, b1r, dsum):
    return pl.pallas_call(
        _tc1b_body,
        grid=(_NRB,),
        in_specs=[
            pl.BlockSpec((_RB, D_IN), lambda i: (i, 0)),
            pl.BlockSpec((D_IN, D_HID), lambda i: (0, 0)),
            pl.BlockSpec((1, D_HID), lambda i: (0, 0)),
            pl.BlockSpec((NC, 8, DDEG), lambda i: (0, 0, 0)),
        ],
        out_specs=pl.BlockSpec((_RB, D_HID), lambda i: (i, 0)),
        out_shape=jax.ShapeDtypeStruct((N_NODES, D_HID), jnp.float32),
    )(x, w1r, b1r, dsum)


def _tc2a_body(p1_ref, ds_ref, r1_ref, b1l_ref, w2l_ref, h_ref, z2_ref):
    agg = (p1_ref[0].astype(jnp.float32)
           + p1_ref[1].astype(jnp.float32))
    ds = ds_ref[0] + ds_ref[1]
    deg = ds[:, 0:1]
    invd = 1.0 / jnp.maximum(deg, 1.0)
    h = jnp.maximum(agg * invd + b1l_ref[...] + r1_ref[...], 0.0)
    h_ref[...] = h
    z2 = jnp.dot(h.astype(jnp.bfloat16), w2l_ref[...].astype(jnp.bfloat16),
                 preferred_element_type=jnp.float32)
    z2_ref[...] = z2.astype(jnp.bfloat16)


def _tc2a(p1, dsum, r1, b1l, w2l):
    return pl.pallas_call(
        _tc2a_body,
        grid=(_NRB,),
        in_specs=[
            pl.BlockSpec((NC, _RB, D_HID), lambda i: (0, i, 0)),
            pl.BlockSpec((NC, _RB, DDEG), lambda i: (0, i, 0)),
            pl.BlockSpec((_RB, D_HID), lambda i: (i, 0)),
            pl.BlockSpec((1, D_HID), lambda i: (0, 0)),
            pl.BlockSpec((D_HID, D_OUT), lambda i: (0, 0)),
        ],
        out_specs=[
            pl.BlockSpec((_RB, D_HID), lambda i: (i, 0)),
            pl.BlockSpec((_RB, D_OUT), lambda i: (i, 0)),
        ],
        out_shape=[
            jax.ShapeDtypeStruct((N_NODES, D_HID), jnp.float32),
            jax.ShapeDtypeStruct((N_NODES, D_OUT), jnp.bfloat16),
        ],
    )(p1, dsum, r1, b1l, w2l)


def _tc2b_body(h_ref, ds_ref, w2r_ref, b2c_ref, r2p_ref):
    ds = ds_ref[0] + ds_ref[1]
    deg = ds[:, 0:1]
    invd = 1.0 / jnp.maximum(deg, 1.0)
    r2c = (jnp.dot(h_ref[...].astype(jnp.bfloat16),
                   w2r_ref[...].astype(jnp.bfloat16),
                   preferred_element_type=jnp.float32) + b2c_ref[...])
    pad = jnp.zeros((_RB, D2P - D_OUT - 1), jnp.float32)
    r2p_ref[...] = jnp.concatenate([r2c, invd, pad], axis=1)


def _tc2b(h, dsum, w2r, b2c):
    return pl.pallas_call(
        _tc2b_body,
        grid=(_NRB,),
        in_specs=[
            pl.BlockSpec((_RB, D_HID), lambda i: (i, 0)),
            pl.BlockSpec((NC, _RB, DDEG), lambda i: (0, i, 0)),
            pl.BlockSpec((D_HID, D_OUT), lambda i: (0, 0)),
            pl.BlockSpec((1, D_OUT), lambda i: (0, 0)),
        ],
        out_specs=pl.BlockSpec((_RB, D2P), lambda i: (i, 0)),
        out_shape=jax.ShapeDtypeStruct((N_NODES, D2P), jnp.float32),
    )(h, dsum, w2r, b2c)


def _tc3_body(p2_ref, r2p_ref, y_ref, m_ref, out_ref, num_ref, den_ref):
    i = pl.program_id(0)

    agg2 = (p2_ref[0].astype(jnp.float32)
            + p2_ref[1].astype(jnp.float32))
    r2c = r2p_ref[:, :D_OUT]
    invd = r2p_ref[:, D_OUT:D_OUT + 1]
    logits = agg2 * invd + r2c
    mx = jnp.max(logits, axis=1, keepdims=True)
    lse = jnp.log(jnp.sum(jnp.exp(logits - mx), axis=1, keepdims=True))
    lsm = logits - mx - lse
    onehot = (lax.broadcasted_iota(jnp.int32, (_RB, D_OUT), 1)
              == y_ref[...]).astype(jnp.float32)
    picked = jnp.sum(lsm * onehot, axis=1, keepdims=True)
    m = m_ref[...]
    num_p = jnp.sum(picked * m)
    den_p = jnp.sum(m)

    @pl.when(i == 0)
    def _():
        num_ref[0] = num_p
        den_ref[0] = den_p

    @pl.when(i > 0)
    def _():
        num_ref[0] = num_ref[0] + num_p
        den_ref[0] = den_ref[0] + den_p

    @pl.when(i == _NRB - 1)
    def _():
        loss = -num_ref[0] / jnp.maximum(den_ref[0], 1.0)
        out_ref[...] = jnp.broadcast_to(loss, (1, 1))


def _tc3(p2, r2p, y2d, m2d):
    return pl.pallas_call(
        _tc3_body,
        grid=(_NRB,),
        in_specs=[
            pl.BlockSpec((NC, _RB, D_OUT), lambda i: (0, i, 0)),
            pl.BlockSpec((_RB, D2P), lambda i: (i, 0)),
            pl.BlockSpec((_RB, 1), lambda i: (i, 0)),
            pl.BlockSpec((_RB, 1), lambda i: (i, 0)),
        ],
        out_specs=pl.BlockSpec((1, 1), lambda i: (0, 0)),
        out_shape=jax.ShapeDtypeStruct((1, 1), jnp.float32),
        scratch_shapes=[
            pltpu.SMEM((1,), jnp.float32),
            pltpu.SMEM((1,), jnp.float32),
        ],
    )(p2, r2p, y2d, m2d)


def kernel(x, edge_index, y, train_mask,
           W1_l, b1_l, W1_r, b1_r, W2_l, b2_l, W2_r, b2_r):
    ei = edge_index.reshape(2, NW, _N_CHUNKS, _CHUNK)
    zinit1 = jnp.zeros((ROWS_PER_TILE, D_HID), jnp.bfloat16)
    zinit2 = jnp.zeros((ROWS_PER_TILE, D_OUT), jnp.bfloat16)
    zinitd = jnp.zeros((ROWS_PER_TILE, DDEG), jnp.float32)
    onesd = jnp.ones((_CHUNK, DDEG), jnp.float32)

    dsum = _sc_deg(ei, onesd, zinitd)
    z1 = _tc1a(x, W1_l)
    p1 = _sc_agg1(z1, ei, zinit1)
    r1 = _tc1b(x, W1_r, b1_r.reshape(1, D_HID), dsum)
    h, z2 = _tc2a(p1, dsum, r1, b1_l.reshape(1, D_HID), W2_l)
    p2 = _sc_agg2(z2, ei, zinit2)
    b2c = (b2_l + b2_r).reshape(1, D_OUT)
    r2p = _tc2b(h, dsum, W2_r, b2c)
    loss = _tc3(p2, r2p, y.reshape(N_NODES, 1).astype(jnp.int32),
                train_mask.reshape(N_NODES, 1).astype(jnp.float32))
    return loss.reshape(1)


# speedup vs baseline: 1.1427x; 1.0060x over previous
import functools

import jax
import jax.numpy as jnp
from jax import lax
from jax.experimental import pallas as pl
from jax.experimental.pallas import tpu as pltpu
from jax.experimental.pallas import tpu_sc as plsc

N_NODES = 10000
N_EDGES = 320000
D_IN = 128
D_HID = 128
D_OUT = 64

NC = 2
NS = 16
NW = NC * NS
E_PER_TILE = N_EDGES // NW
N_PAD = 10240
ROWS_PER_TILE = N_PAD // NS

D2P = D_OUT + 8
DDEG = 8

NBUF = 5

_sc_mesh = plsc.VectorSubcoreMesh(core_axis_name="c", subcore_axis_name="s")
_sc_params = pltpu.CompilerParams(use_tc_tiling_on_sc=False)


def _make_sc_agg(d, chunk, nbuf=NBUF):
    n_chunks = E_PER_TILE // chunk
    n_groups = n_chunks // nbuf

    @functools.partial(
        pl.kernel,
        mesh=_sc_mesh,
        compiler_params=_sc_params,
        out_type=jax.ShapeDtypeStruct((NC, N_PAD, d), jnp.bfloat16),
        scratch_types=[
            pltpu.VMEM((n_chunks, chunk), jnp.int32),
            pltpu.VMEM((n_chunks, chunk), jnp.int32),
            pltpu.VMEM((nbuf, chunk, d), jnp.bfloat16),
            pltpu.VMEM_SHARED((N_PAD, d), jnp.bfloat16),
            pltpu.SemaphoreType.DMA((nbuf,)),
            pltpu.SemaphoreType.DMA((nbuf,)),
        ],
    )
    def sc_agg(z_hbm, ei_hbm, zinit_hbm, out_hbm,
               src_v, dst_v, rows_v, acc_sh, gsem, ssem):
        cid = lax.axis_index("c")
        sid = lax.axis_index("s")
        wid = cid * NS + sid

        pltpu.sync_copy(ei_hbm.at[0, wid], src_v)
        pltpu.sync_copy(ei_hbm.at[1, wid], dst_v)
        pltpu.sync_copy(zinit_hbm,
                        acc_sh.at[pl.ds(sid * ROWS_PER_TILE, ROWS_PER_TILE), :])
        plsc.subcore_barrier()

        for t in range(nbuf):
            pltpu.async_copy(z_hbm.at[src_v.at[t]], rows_v.at[t], gsem.at[t])

        def body(g, carry):
            j0 = g * nbuf
            for t in range(nbuf):
                j = j0 + t
                pltpu.make_async_copy(z_hbm.at[src_v.at[j]], rows_v.at[t],
                                      gsem.at[t]).wait()
                pltpu.async_copy(rows_v.at[t], acc_sh.at[dst_v.at[j]],
                                 ssem.at[t], add=True)
            for t in range(nbuf):
                j = j0 + t
                jn = j + nbuf
                pltpu.make_async_copy(rows_v.at[t], acc_sh.at[dst_v.at[j]],
                                      ssem.at[t]).wait()

                @pl.when(jn < n_chunks)
                def _():
                    pltpu.async_copy(z_hbm.at[src_v.at[jn]], rows_v.at[t],
                                     gsem.at[t])
            return carry

        lax.fori_loop(0, n_groups, body, 0)
        plsc.subcore_barrier()

        pltpu.sync_copy(acc_sh.at[pl.ds(sid * ROWS_PER_TILE, ROWS_PER_TILE), :],
                        out_hbm.at[cid, pl.ds(sid * ROWS_PER_TILE, ROWS_PER_TILE), :])

    return sc_agg


_CHUNK = 80
_N_CHUNKS = E_PER_TILE // _CHUNK
_N_GROUPS = _N_CHUNKS // NBUF
_sc_agg1 = _make_sc_agg(D_HID, _CHUNK)
_sc_agg2 = _make_sc_agg(D_OUT, _CHUNK)


@functools.partial(
    pl.kernel,
    mesh=_sc_mesh,
    compiler_params=_sc_params,
    out_type=jax.ShapeDtypeStruct((NC, N_PAD, DDEG), jnp.float32),
    scratch_types=[
        pltpu.VMEM((_N_CHUNKS, _CHUNK), jnp.int32),
        pltpu.VMEM((_CHUNK, DDEG), jnp.float32),
        pltpu.VMEM_SHARED((N_PAD, DDEG), jnp.float32),
        pltpu.SemaphoreType.DMA((NBUF,)),
    ],
)
def _sc_deg(ei_hbm, ones_hbm, zinit_hbm, out_hbm, dst_v, ones_v, acc_sh, ssem):
    cid = lax.axis_index("c")
    sid = lax.axis_index("s")
    wid = cid * NS + sid

    pltpu.sync_copy(ei_hbm.at[1, wid], dst_v)
    pltpu.sync_copy(ones_hbm, ones_v)
    pltpu.sync_copy(zinit_hbm,
                    acc_sh.at[pl.ds(sid * ROWS_PER_TILE, ROWS_PER_TILE), :])
    plsc.subcore_barrier()

    def body(g, carry):
        j0 = g * NBUF
        for t in range(NBUF):
            pltpu.async_copy(ones_v, acc_sh.at[dst_v.at[j0 + t]],
                             ssem.at[t], add=True)
        for t in range(NBUF):
            pltpu.make_async_copy(ones_v, acc_sh.at[dst_v.at[j0 + t]],
                                  ssem.at[t]).wait()
        return carry

    lax.fori_loop(0, _N_GROUPS, body, 0)
    plsc.subcore_barrier()

    pltpu.sync_copy(acc_sh.at[pl.ds(sid * ROWS_PER_TILE, ROWS_PER_TILE), :],
                    out_hbm.at[cid, pl.ds(sid * ROWS_PER_TILE, ROWS_PER_TILE), :])


_RB = 2000
_NRB = N_NODES // _RB


def _tc1a_body(x_ref, w1l_ref, z1_ref):
    z1 = jnp.dot(x_ref[...].astype(jnp.bfloat16),
                 w1l_ref[...].astype(jnp.bfloat16),
                 preferred_element_type=jnp.float32)
    z1_ref[...] = z1.astype(jnp.bfloat16)


def _tc1a(x, w1l):
    return pl.pallas_call(
        _tc1a_body,
        grid=(_NRB,),
        in_specs=[
            pl.BlockSpec((_RB, D_IN), lambda i: (i, 0)),
            pl.BlockSpec((D_IN, D_HID), lambda i: (0, 0)),
        ],
        out_specs=pl.BlockSpec((_RB, D_HID), lambda i: (i, 0)),
        out_shape=jax.ShapeDtypeStruct((N_NODES, D_HID), jnp.bfloat16),
    )(x, w1l)


def _tc1b_body(x_ref, w1r_ref, b1r_ref, r1_ref):
    r1_ref[...] = (jnp.dot(x_ref[...].astype(jnp.bfloat16),
                           w1r_ref[...].astype(jnp.bfloat16),
                           preferred_element_type=jnp.float32) + b1r_ref[...])


def _tc1b(x, w1r, b1r):
    return pl.pallas_call(
        _tc1b_body,
        grid=(_NRB,),
        in_specs=[
            pl.BlockSpec((_RB, D_IN), lambda i: (i, 0)),
            pl.BlockSpec((D_IN, D_HID), lambda i: (0, 0)),
            pl.BlockSpec((1, D_HID), lambda i: (0, 0)),
        ],
        out_specs=pl.BlockSpec((_RB, D_HID), lambda i: (i, 0)),
        out_shape=jax.ShapeDtypeStruct((N_NODES, D_HID), jnp.float32),
    )(x, w1r, b1r)


def _tc2a_body(p1_ref, ds_ref, r1_ref, b1l_ref, w2l_ref, h_ref, z2_ref):
    agg = (p1_ref[0].astype(jnp.float32)
           + p1_ref[1].astype(jnp.float32))
    ds = ds_ref[0] + ds_ref[1]
    deg = ds[:, 0:1]
    invd = 1.0 / jnp.maximum(deg, 1.0)
    h = jnp.maximum(agg * invd + b1l_ref[...] + r1_ref[...], 0.0)
    h_ref[...] = h
    z2 = jnp.dot(h.astype(jnp.bfloat16), w2l_ref[...].astype(jnp.bfloat16),
                 preferred_element_type=jnp.float32)
    z2_ref[...] = z2.astype(jnp.bfloat16)


def _tc2a(p1, dsum, r1, b1l, w2l):
    return pl.pallas_call(
        _tc2a_body,
        grid=(_NRB,),
        in_specs=[
            pl.BlockSpec((NC, _RB, D_HID), lambda i: (0, i, 0)),
            pl.BlockSpec((NC, _RB, DDEG), lambda i: (0, i, 0)),
            pl.BlockSpec((_RB, D_HID), lambda i: (i, 0)),
            pl.BlockSpec((1, D_HID), lambda i: (0, 0)),
            pl.BlockSpec((D_HID, D_OUT), lambda i: (0, 0)),
        ],
        out_specs=[
            pl.BlockSpec((_RB, D_HID), lambda i: (i, 0)),
            pl.BlockSpec((_RB, D_OUT), lambda i: (i, 0)),
        ],
        out_shape=[
            jax.ShapeDtypeStruct((N_NODES, D_HID), jnp.float32),
            jax.ShapeDtypeStruct((N_NODES, D_OUT), jnp.bfloat16),
        ],
    )(p1, dsum, r1, b1l, w2l)


def _tc2b_body(h_ref, ds_ref, w2r_ref, b2c_ref, r2p_ref):
    ds = ds_ref[0] + ds_ref[1]
    deg = ds[:, 0:1]
    invd = 1.0 / jnp.maximum(deg, 1.0)
    r2c = (jnp.dot(h_ref[...].astype(jnp.bfloat16),
                   w2r_ref[...].astype(jnp.bfloat16),
                   preferred_element_type=jnp.float32) + b2c_ref[...])
    pad = jnp.zeros((_RB, D2P - D_OUT - 1), jnp.float32)
    r2p_ref[...] = jnp.concatenate([r2c, invd, pad], axis=1)


def _tc2b(h, dsum, w2r, b2c):
    return pl.pallas_call(
        _tc2b_body,
        grid=(_NRB,),
        in_specs=[
            pl.BlockSpec((_RB, D_HID), lambda i: (i, 0)),
            pl.BlockSpec((NC, _RB, DDEG), lambda i: (0, i, 0)),
            pl.BlockSpec((D_HID, D_OUT), lambda i: (0, 0)),
            pl.BlockSpec((1, D_OUT), lambda i: (0, 0)),
        ],
        out_specs=pl.BlockSpec((_RB, D2P), lambda i: (i, 0)),
        out_shape=jax.ShapeDtypeStruct((N_NODES, D2P), jnp.float32),
    )(h, dsum, w2r, b2c)


def _tc3_body(p2_ref, r2p_ref, y_ref, m_ref, out_ref, num_ref, den_ref):
    i = pl.program_id(0)

    agg2 = (p2_ref[0].astype(jnp.float32)
            + p2_ref[1].astype(jnp.float32))
    r2c = r2p_ref[:, :D_OUT]
    invd = r2p_ref[:, D_OUT:D_OUT + 1]
    logits = agg2 * invd + r2c
    mx = jnp.max(logits, axis=1, keepdims=True)
    lse = jnp.log(jnp.sum(jnp.exp(logits - mx), axis=1, keepdims=True))
    lsm = logits - mx - lse
    onehot = (lax.broadcasted_iota(jnp.int32, (_RB, D_OUT), 1)
              == y_ref[...]).astype(jnp.float32)
    picked = jnp.sum(lsm * onehot, axis=1, keepdims=True)
    m = m_ref[...]
    num_p = jnp.sum(picked * m)
    den_p = jnp.sum(m)

    @pl.when(i == 0)
    def _():
        num_ref[0] = num_p
        den_ref[0] = den_p

    @pl.when(i > 0)
    def _():
        num_ref[0] = num_ref[0] + num_p
        den_ref[0] = den_ref[0] + den_p

    @pl.when(i == _NRB - 1)
    def _():
        loss = -num_ref[0] / jnp.maximum(den_ref[0], 1.0)
        out_ref[...] = jnp.broadcast_to(loss, (1, 1))


def _tc3(p2, r2p, y2d, m2d):
    return pl.pallas_call(
        _tc3_body,
        grid=(_NRB,),
        in_specs=[
            pl.BlockSpec((NC, _RB, D_OUT), lambda i: (0, i, 0)),
            pl.BlockSpec((_RB, D2P), lambda i: (i, 0)),
            pl.BlockSpec((_RB, 1), lambda i: (i, 0)),
            pl.BlockSpec((_RB, 1), lambda i: (i, 0)),
        ],
        out_specs=pl.BlockSpec((1, 1), lambda i: (0, 0)),
        out_shape=jax.ShapeDtypeStruct((1, 1), jnp.float32),
        scratch_shapes=[
            pltpu.SMEM((1,), jnp.float32),
            pltpu.SMEM((1,), jnp.float32),
        ],
    )(p2, r2p, y2d, m2d)


def kernel(x, edge_index, y, train_mask,
           W1_l, b1_l, W1_r, b1_r, W2_l, b2_l, W2_r, b2_r):
    ei = edge_index.reshape(2, NW, _N_CHUNKS, _CHUNK)
    zinit1 = jnp.zeros((ROWS_PER_TILE, D_HID), jnp.bfloat16)
    zinit2 = jnp.zeros((ROWS_PER_TILE, D_OUT), jnp.bfloat16)
    zinitd = jnp.zeros((ROWS_PER_TILE, DDEG), jnp.float32)
    onesd = jnp.ones((_CHUNK, DDEG), jnp.float32)

    dsum = _sc_deg(ei, onesd, zinitd)
    z1 = _tc1a(x, W1_l)
    p1 = _sc_agg1(z1, ei, zinit1)
    r1 = _tc1b(x, W1_r, b1_r.reshape(1, D_HID))
    h, z2 = _tc2a(p1, dsum, r1, b1_l.reshape(1, D_HID), W2_l)
    p2 = _sc_agg2(z2, ei, zinit2)
    b2c = (b2_l + b2_r).reshape(1, D_OUT)
    r2p = _tc2b(h, dsum, W2_r, b2c)
    loss = _tc3(p2, r2p, y.reshape(N_NODES, 1).astype(jnp.int32),
                train_mask.reshape(N_NODES, 1).astype(jnp.float32))
    return loss.reshape(1)


# speedup vs baseline: 1.1568x; 1.0123x over previous
import functools

import jax
import jax.numpy as jnp
from jax import lax
from jax.experimental import pallas as pl
from jax.experimental.pallas import tpu as pltpu
from jax.experimental.pallas import tpu_sc as plsc

N_NODES = 10000
N_EDGES = 320000
D_IN = 128
D_HID = 128
D_OUT = 64

NC = 2
NS = 16
NW = NC * NS
E_PER_TILE = N_EDGES // NW
N_PAD = 10240
ROWS_PER_TILE = N_PAD // NS

D2P = D_OUT + 8
DDEG = 8

NBUF = 5

_sc_mesh = plsc.VectorSubcoreMesh(core_axis_name="c", subcore_axis_name="s")
_sc_params = pltpu.CompilerParams(use_tc_tiling_on_sc=False)


def _make_sc_agg(d, chunk, nbuf=NBUF):
    n_chunks = E_PER_TILE // chunk
    n_groups = n_chunks // nbuf

    @functools.partial(
        pl.kernel,
        mesh=_sc_mesh,
        compiler_params=_sc_params,
        out_type=jax.ShapeDtypeStruct((NC, N_PAD, d), jnp.bfloat16),
        scratch_types=[
            pltpu.VMEM((n_chunks, chunk), jnp.int32),
            pltpu.VMEM((n_chunks, chunk), jnp.int32),
            pltpu.VMEM((nbuf, chunk, d), jnp.bfloat16),
            pltpu.VMEM_SHARED((N_PAD, d), jnp.bfloat16),
            pltpu.SemaphoreType.DMA((nbuf,)),
            pltpu.SemaphoreType.DMA((nbuf,)),
        ],
    )
    def sc_agg(z_hbm, ei_hbm, zinit_hbm, out_hbm,
               src_v, dst_v, rows_v, acc_sh, gsem, ssem):
        cid = lax.axis_index("c")
        sid = lax.axis_index("s")
        wid = cid * NS + sid

        pltpu.sync_copy(ei_hbm.at[0, wid], src_v)
        pltpu.sync_copy(ei_hbm.at[1, wid], dst_v)
        pltpu.sync_copy(zinit_hbm,
                        acc_sh.at[pl.ds(sid * ROWS_PER_TILE, ROWS_PER_TILE), :])
        plsc.subcore_barrier()

        for t in range(nbuf):
            pltpu.async_copy(z_hbm.at[src_v.at[t]], rows_v.at[t], gsem.at[t])

        def body(g, carry):
            j0 = g * nbuf
            for t in range(nbuf):
                j = j0 + t
                pltpu.make_async_copy(z_hbm.at[src_v.at[j]], rows_v.at[t],
                                      gsem.at[t]).wait()
                pltpu.async_copy(rows_v.at[t], acc_sh.at[dst_v.at[j]],
                                 ssem.at[t], add=True)
            for t in range(nbuf):
                j = j0 + t
                jn = j + nbuf
                pltpu.make_async_copy(rows_v.at[t], acc_sh.at[dst_v.at[j]],
                                      ssem.at[t]).wait()

                @pl.when(jn < n_chunks)
                def _():
                    pltpu.async_copy(z_hbm.at[src_v.at[jn]], rows_v.at[t],
                                     gsem.at[t])
            return carry

        lax.fori_loop(0, n_groups, body, 0)
        plsc.subcore_barrier()

        pltpu.sync_copy(acc_sh.at[pl.ds(sid * ROWS_PER_TILE, ROWS_PER_TILE), :],
                        out_hbm.at[cid, pl.ds(sid * ROWS_PER_TILE, ROWS_PER_TILE), :])

    return sc_agg


_CHUNK = 80
_N_CHUNKS = E_PER_TILE // _CHUNK
_N_GROUPS = _N_CHUNKS // NBUF
_sc_agg1 = _make_sc_agg(D_HID, _CHUNK)
_sc_agg2 = _make_sc_agg(D_OUT, _CHUNK)


@functools.partial(
    pl.kernel,
    mesh=_sc_mesh,
    compiler_params=_sc_params,
    out_type=jax.ShapeDtypeStruct((NC, N_PAD, DDEG), jnp.float32),
    scratch_types=[
        pltpu.VMEM((_N_CHUNKS, _CHUNK), jnp.int32),
        pltpu.VMEM((_CHUNK, DDEG), jnp.float32),
        pltpu.VMEM_SHARED((N_PAD, DDEG), jnp.float32),
        pltpu.SemaphoreType.DMA((NBUF,)),
    ],
)
def _sc_deg(ei_hbm, ones_hbm, zinit_hbm, out_hbm, dst_v, ones_v, acc_sh, ssem):
    cid = lax.axis_index("c")
    sid = lax.axis_index("s")
    wid = cid * NS + sid

    pltpu.sync_copy(ei_hbm.at[1, wid], dst_v)
    pltpu.sync_copy(ones_hbm, ones_v)
    pltpu.sync_copy(zinit_hbm,
                    acc_sh.at[pl.ds(sid * ROWS_PER_TILE, ROWS_PER_TILE), :])
    plsc.subcore_barrier()

    def body(g, carry):
        j0 = g * NBUF
        for t in range(NBUF):
            pltpu.async_copy(ones_v, acc_sh.at[dst_v.at[j0 + t]],
                             ssem.at[t], add=True)
        for t in range(NBUF):
            pltpu.make_async_copy(ones_v, acc_sh.at[dst_v.at[j0 + t]],
                                  ssem.at[t]).wait()
        return carry

    lax.fori_loop(0, _N_GROUPS, body, 0)
    plsc.subcore_barrier()

    pltpu.sync_copy(acc_sh.at[pl.ds(sid * ROWS_PER_TILE, ROWS_PER_TILE), :],
                    out_hbm.at[cid, pl.ds(sid * ROWS_PER_TILE, ROWS_PER_TILE), :])


_RB = 5000
_NRB = N_NODES // _RB


def _tc1a_body(x_ref, w1l_ref, z1_ref):
    z1 = jnp.dot(x_ref[...].astype(jnp.bfloat16),
                 w1l_ref[...].astype(jnp.bfloat16),
                 preferred_element_type=jnp.float32)
    z1_ref[...] = z1.astype(jnp.bfloat16)


def _tc1a(x, w1l):
    return pl.pallas_call(
        _tc1a_body,
        grid=(_NRB,),
        in_specs=[
            pl.BlockSpec((_RB, D_IN), lambda i: (i, 0)),
            pl.BlockSpec((D_IN, D_HID), lambda i: (0, 0)),
        ],
        out_specs=pl.BlockSpec((_RB, D_HID), lambda i: (i, 0)),
        out_shape=jax.ShapeDtypeStruct((N_NODES, D_HID), jnp.bfloat16),
    )(x, w1l)


def _tc1b_body(x_ref, w1r_ref, b1r_ref, r1_ref):
    r1_ref[...] = (jnp.dot(x_ref[...].astype(jnp.bfloat16),
                           w1r_ref[...].astype(jnp.bfloat16),
                           preferred_element_type=jnp.float32) + b1r_ref[...])


def _tc1b(x, w1r, b1r):
    return pl.pallas_call(
        _tc1b_body,
        grid=(_NRB,),
        in_specs=[
            pl.BlockSpec((_RB, D_IN), lambda i: (i, 0)),
            pl.BlockSpec((D_IN, D_HID), lambda i: (0, 0)),
            pl.BlockSpec((1, D_HID), lambda i: (0, 0)),
        ],
        out_specs=pl.BlockSpec((_RB, D_HID), lambda i: (i, 0)),
        out_shape=jax.ShapeDtypeStruct((N_NODES, D_HID), jnp.float32),
    )(x, w1r, b1r)


def _tc2a_body(p1_ref, ds_ref, r1_ref, b1l_ref, w2l_ref, h_ref, z2_ref):
    agg = (p1_ref[0].astype(jnp.float32)
           + p1_ref[1].astype(jnp.float32))
    ds = ds_ref[0] + ds_ref[1]
    deg = ds[:, 0:1]
    invd = 1.0 / jnp.maximum(deg, 1.0)
    h = jnp.maximum(agg * invd + b1l_ref[...] + r1_ref[...], 0.0)
    h_ref[...] = h
    z2 = jnp.dot(h.astype(jnp.bfloat16), w2l_ref[...].astype(jnp.bfloat16),
                 preferred_element_type=jnp.float32)
    z2_ref[...] = z2.astype(jnp.bfloat16)


def _tc2a(p1, dsum, r1, b1l, w2l):
    return pl.pallas_call(
        _tc2a_body,
        grid=(_NRB,),
        in_specs=[
            pl.BlockSpec((NC, _RB, D_HID), lambda i: (0, i, 0)),
            pl.BlockSpec((NC, _RB, DDEG), lambda i: (0, i, 0)),
            pl.BlockSpec((_RB, D_HID), lambda i: (i, 0)),
            pl.BlockSpec((1, D_HID), lambda i: (0, 0)),
            pl.BlockSpec((D_HID, D_OUT), lambda i: (0, 0)),
        ],
        out_specs=[
            pl.BlockSpec((_RB, D_HID), lambda i: (i, 0)),
            pl.BlockSpec((_RB, D_OUT), lambda i: (i, 0)),
        ],
        out_shape=[
            jax.ShapeDtypeStruct((N_NODES, D_HID), jnp.float32),
            jax.ShapeDtypeStruct((N_NODES, D_OUT), jnp.bfloat16),
        ],
    )(p1, dsum, r1, b1l, w2l)


def _tc2b_body(h_ref, ds_ref, w2r_ref, b2c_ref, r2p_ref):
    ds = ds_ref[0] + ds_ref[1]
    deg = ds[:, 0:1]
    invd = 1.0 / jnp.maximum(deg, 1.0)
    r2c = (jnp.dot(h_ref[...].astype(jnp.bfloat16),
                   w2r_ref[...].astype(jnp.bfloat16),
                   preferred_element_type=jnp.float32) + b2c_ref[...])
    pad = jnp.zeros((_RB, D2P - D_OUT - 1), jnp.float32)
    r2p_ref[...] = jnp.concatenate([r2c, invd, pad], axis=1)


def _tc2b(h, dsum, w2r, b2c):
    return pl.pallas_call(
        _tc2b_body,
        grid=(_NRB,),
        in_specs=[
            pl.BlockSpec((_RB, D_HID), lambda i: (i, 0)),
            pl.BlockSpec((NC, _RB, DDEG), lambda i: (0, i, 0)),
            pl.BlockSpec((D_HID, D_OUT), lambda i: (0, 0)),
            pl.BlockSpec((1, D_OUT), lambda i: (0, 0)),
        ],
        out_specs=pl.BlockSpec((_RB, D2P), lambda i: (i, 0)),
        out_shape=jax.ShapeDtypeStruct((N_NODES, D2P), jnp.float32),
    )(h, dsum, w2r, b2c)


def _tc3_body(p2_ref, r2p_ref, y_ref, m_ref, out_ref, num_ref, den_ref):
    i = pl.program_id(0)

    agg2 = (p2_ref[0].astype(jnp.float32)
            + p2_ref[1].astype(jnp.float32))
    r2c = r2p_ref[:, :D_OUT]
    invd = r2p_ref[:, D_OUT:D_OUT + 1]
    logits = agg2 * invd + r2c
    mx = jnp.max(logits, axis=1, keepdims=True)
    lse = jnp.log(jnp.sum(jnp.exp(logits - mx), axis=1, keepdims=True))
    lsm = logits - mx - lse
    onehot = (lax.broadcasted_iota(jnp.int32, (_RB, D_OUT), 1)
              == y_ref[...]).astype(jnp.float32)
    picked = jnp.sum(lsm * onehot, axis=1, keepdims=True)
    m = m_ref[...]
    num_p = jnp.sum(picked * m)
    den_p = jnp.sum(m)

    @pl.when(i == 0)
    def _():
        num_ref[0] = num_p
        den_ref[0] = den_p

    @pl.when(i > 0)
    def _():
        num_ref[0] = num_ref[0] + num_p
        den_ref[0] = den_ref[0] + den_p

    @pl.when(i == _NRB - 1)
    def _():
        loss = -num_ref[0] / jnp.maximum(den_ref[0], 1.0)
        out_ref[...] = jnp.broadcast_to(loss, (1, 1))


def _tc3(p2, r2p, y2d, m2d):
    return pl.pallas_call(
        _tc3_body,
        grid=(_NRB,),
        in_specs=[
            pl.BlockSpec((NC, _RB, D_OUT), lambda i: (0, i, 0)),
            pl.BlockSpec((_RB, D2P), lambda i: (i, 0)),
            pl.BlockSpec((_RB, 1), lambda i: (i, 0)),
            pl.BlockSpec((_RB, 1), lambda i: (i, 0)),
        ],
        out_specs=pl.BlockSpec((1, 1), lambda i: (0, 0)),
        out_shape=jax.ShapeDtypeStruct((1, 1), jnp.float32),
        scratch_shapes=[
            pltpu.SMEM((1,), jnp.float32),
            pltpu.SMEM((1,), jnp.float32),
        ],
    )(p2, r2p, y2d, m2d)


def kernel(x, edge_index, y, train_mask,
           W1_l, b1_l, W1_r, b1_r, W2_l, b2_l, W2_r, b2_r):
    ei = edge_index.reshape(2, NW, _N_CHUNKS, _CHUNK)
    zinit1 = jnp.zeros((ROWS_PER_TILE, D_HID), jnp.bfloat16)
    zinit2 = jnp.zeros((ROWS_PER_TILE, D_OUT), jnp.bfloat16)
    zinitd = jnp.zeros((ROWS_PER_TILE, DDEG), jnp.float32)
    onesd = jnp.ones((_CHUNK, DDEG), jnp.float32)

    dsum = _sc_deg(ei, onesd, zinitd)
    z1 = _tc1a(x, W1_l)
    p1 = _sc_agg1(z1, ei, zinit1)
    r1 = _tc1b(x, W1_r, b1_r.reshape(1, D_HID))
    h, z2 = _tc2a(p1, dsum, r1, b1_l.reshape(1, D_HID), W2_l)
    p2 = _sc_agg2(z2, ei, zinit2)
    b2c = (b2_l + b2_r).reshape(1, D_OUT)
    r2p = _tc2b(h, dsum, W2_r, b2c)
    loss = _tc3(p2, r2p, y.reshape(N_NODES, 1).astype(jnp.int32),
                train_mask.reshape(N_NODES, 1).astype(jnp.float32))
    return loss.reshape(1)
